# baseline (speedup 1.0000x reference)
import jax
import jax.numpy as jnp
from jax.experimental import pallas as pl
from jax.experimental.pallas import tpu as pltpu

_BN = 1000
_KC_OFF = 0
_KB_OFF = 128


def _fused_linears_kernel(x_ref, wc_hbm, bc_hbm, wb_hbm, bb_hbm,
                          scores_ref, deltas_ref,
                          wcat_v, wc_v, wb_v, bc_v, bb_v, wsem):
    i = pl.program_id(0)
    kc = wc_hbm.shape[1]
    kb = wb_hbm.shape[1]

    @pl.when(i == 0)
    def _load_weights():
        copies = [
            pltpu.make_async_copy(wc_hbm, wc_v, wsem.at[0]),
            pltpu.make_async_copy(wb_hbm, wb_v, wsem.at[1]),
            pltpu.make_async_copy(bc_hbm, bc_v, wsem.at[2]),
            pltpu.make_async_copy(bb_hbm, bb_v, wsem.at[3]),
        ]
        for c in copies:
            c.start()
        for c in copies:
            c.wait()
        wcat_v[:, _KC_OFF:_KC_OFF + kc] = wc_v[...]
        wcat_v[:, _KB_OFF:_KB_OFF + kb] = wb_v[...]

    r = jnp.dot(x_ref[...], wcat_v[...],
                precision=jax.lax.Precision.DEFAULT,
                preferred_element_type=jnp.float32)
    scores_ref[...] = r[:, _KC_OFF:_KC_OFF + kc] + bc_v[...]
    deltas_ref[...] = r[:, _KB_OFF:_KB_OFF + kb] + bb_v[...]


@jax.jit
def kernel(x, W_cls, b_cls, W_box, b_box):
    if x.ndim > 2:
        x = x.reshape((x.shape[0], -1))
    n, d = x.shape
    kc = W_cls.shape[1]
    kb = W_box.shape[1]
    bn = _BN if n % _BN == 0 else n
    scores, deltas = pl.pallas_call(
        _fused_linears_kernel,
        grid=(n // bn,),
        in_specs=[
            pl.BlockSpec((bn, d), lambda i: (i, 0)),
            pl.BlockSpec(memory_space=pl.ANY),
            pl.BlockSpec(memory_space=pl.ANY),
            pl.BlockSpec(memory_space=pl.ANY),
            pl.BlockSpec(memory_space=pl.ANY),
        ],
        out_specs=[
            pl.BlockSpec((bn, kc), lambda i: (i, 0)),
            pl.BlockSpec((bn, kb), lambda i: (i, 0)),
        ],
        out_shape=[
            jax.ShapeDtypeStruct((n, kc), jnp.float32),
            jax.ShapeDtypeStruct((n, kb), jnp.float32),
        ],
        scratch_shapes=[
            pltpu.VMEM((d, _KB_OFF + kb), jnp.float32),
            pltpu.VMEM((d, kc), jnp.float32),
            pltpu.VMEM((d, kb), jnp.float32),
            pltpu.VMEM((kc,), jnp.float32),
            pltpu.VMEM((kb,), jnp.float32),
            pltpu.SemaphoreType.DMA((4,)),
        ],
        compiler_params=pltpu.CompilerParams(
            dimension_semantics=("arbitrary",),
        ),
    )(x, W_cls, b_cls, W_box, b_box)
    return (scores, deltas)


# speedup vs baseline: 1.0253x; 1.0253x over previous
import jax
import jax.numpy as jnp
from jax.experimental import pallas as pl
from jax.experimental.pallas import tpu as pltpu

_BN = 2000
_KC_OFF = 0
_KB_OFF = 128


def _fused_linears_kernel(x_ref, wc_hbm, bc_hbm, wb_hbm, bb_hbm,
                          scores_ref, deltas_ref,
                          wcat_v, wc_v, wb_v, bc_v, bb_v, wsem):
    i = pl.program_id(0)
    kc = wc_hbm.shape[1]
    kb = wb_hbm.shape[1]

    @pl.when(i == 0)
    def _load_weights():
        copies = [
            pltpu.make_async_copy(wc_hbm, wc_v, wsem.at[0]),
            pltpu.make_async_copy(wb_hbm, wb_v, wsem.at[1]),
            pltpu.make_async_copy(bc_hbm, bc_v, wsem.at[2]),
            pltpu.make_async_copy(bb_hbm, bb_v, wsem.at[3]),
        ]
        for c in copies:
            c.start()
        for c in copies:
            c.wait()
        wcat_v[:, _KC_OFF:_KC_OFF + kc] = wc_v[...]
        wcat_v[:, _KB_OFF:_KB_OFF + kb] = wb_v[...]

    r = jnp.dot(x_ref[...], wcat_v[...],
                precision=jax.lax.Precision.DEFAULT,
                preferred_element_type=jnp.float32)
    scores_ref[...] = r[:, _KC_OFF:_KC_OFF + kc] + bc_v[...]
    deltas_ref[...] = r[:, _KB_OFF:_KB_OFF + kb] + bb_v[...]


@jax.jit
def kernel(x, W_cls, b_cls, W_box, b_box):
    if x.ndim > 2:
        x = x.reshape((x.shape[0], -1))
    n, d = x.shape
    kc = W_cls.shape[1]
    kb = W_box.shape[1]
    bn = _BN if n % _BN == 0 else n
    scores, deltas = pl.pallas_call(
        _fused_linears_kernel,
        grid=(n // bn,),
        in_specs=[
            pl.BlockSpec((bn, d), lambda i: (i, 0)),
            pl.BlockSpec(memory_space=pl.ANY),
            pl.BlockSpec(memory_space=pl.ANY),
            pl.BlockSpec(memory_space=pl.ANY),
            pl.BlockSpec(memory_space=pl.ANY),
        ],
        out_specs=[
            pl.BlockSpec((bn, kc), lambda i: (i, 0)),
            pl.BlockSpec((bn, kb), lambda i: (i, 0)),
        ],
        out_shape=[
            jax.ShapeDtypeStruct((n, kc), jnp.float32),
            jax.ShapeDtypeStruct((n, kb), jnp.float32),
        ],
        scratch_shapes=[
            pltpu.VMEM((d, _KB_OFF + kb), jnp.float32),
            pltpu.VMEM((d, kc), jnp.float32),
            pltpu.VMEM((d, kb), jnp.float32),
            pltpu.VMEM((kc,), jnp.float32),
            pltpu.VMEM((kb,), jnp.float32),
            pltpu.SemaphoreType.DMA((4,)),
        ],
        compiler_params=pltpu.CompilerParams(
            dimension_semantics=("arbitrary",),
        ),
    )(x, W_cls, b_cls, W_box, b_box)
    return (scores, deltas)
